# Initial kernel scaffold; baseline (speedup 1.0000x reference)
import functools

import jax
import jax.numpy as jnp
import numpy as np
from jax import lax
from jax.experimental import pallas as pl
from jax.experimental.pallas import tpu as pltpu
from jax.experimental.pallas import tpu_sc as plsc

CUTOFF = 10.0
LOG2 = float(np.log(2.0))
PI = float(np.pi)

NC = 2
NS = 16
L = 16
NW = NC * NS
CH = 128


def _lin_body(x_ref, w_ref, o_ref):
    o_ref[...] = lax.dot_general(
        x_ref[...], w_ref[...], (((1,), (1,)), ((), ())),
        preferred_element_type=jnp.float32)


def _filter_body(n_edges, eb, attr_ref, ew_ref, w1_ref, b1_ref, w2_ref, b2_ref,
                 o_ref):
    x = lax.dot_general(attr_ref[...], w1_ref[...], (((1,), (1,)), ((), ())),
                        preferred_element_type=jnp.float32)
    x = x + b1_ref[...]
    x = jnp.maximum(x, 0.0) + jnp.log(1.0 + jnp.exp(-jnp.abs(x))) - LOG2
    x = lax.dot_general(x, w2_ref[...], (((1,), (1,)), ((), ())),
                        preferred_element_type=jnp.float32)
    x = x + b2_ref[...]
    ew = ew_ref[...]
    c = 0.5 * (jnp.cos(ew * (PI / CUTOFF)) + 1.0)
    rows = pl.program_id(0) * eb + lax.broadcasted_iota(jnp.int32, ew.shape, 0)
    c = jnp.where(rows < n_edges, c, 0.0)
    o_ref[...] = x * c


def _out_body(h1_ref, a0_ref, a1_ref, w_ref, b_ref, o_ref):
    s = h1_ref[...] + a0_ref[...] + a1_ref[...]
    o_ref[...] = lax.dot_general(
        s, w_ref[...], (((1,), (1,)), ((), ())),
        preferred_element_type=jnp.float32) + b_ref[...]


def _sc_gather_mul_scatter(n_nodes, d, per_tile,
                           h1_hbm, src_hbm, dst_hbm, w_hbm, out_hbm,
                           idx_s, idx_d, rows_v, w_v, agg_sh, gsem):
    cid = lax.axis_index("c")
    sid = lax.axis_index("s")
    wid = cid * NS + sid
    rows_per = n_nodes // NS
    nsub = d // L

    zero = jnp.zeros((L,), jnp.float32)

    def zb(i, _):
        w_v[i // nsub, pl.ds((i % nsub) * L, L)] = zero
        return 0

    lax.fori_loop(0, CH * nsub, zb, 0)
    base_r = sid * rows_per
    nfull = rows_per // CH
    rem = rows_per - nfull * CH
    for k in range(nfull):
        pltpu.sync_copy(w_v, agg_sh.at[pl.ds(base_r + k * CH, CH)])
    if rem:
        pltpu.sync_copy(w_v.at[pl.ds(0, rem)],
                        agg_sh.at[pl.ds(base_r + nfull * CH, rem)])
    plsc.subcore_barrier()

    def chunk_body(ci, _):
        base = (wid * per_tile + ci) * CH
        pltpu.sync_copy(src_hbm.at[pl.ds(base, CH)], idx_s)
        pltpu.sync_copy(dst_hbm.at[pl.ds(base, CH)], idx_d)
        pltpu.async_copy(h1_hbm.at[idx_s], rows_v, gsem).wait()
        pltpu.sync_copy(w_hbm.at[pl.ds(base, CH)], w_v)

        def emul(e, _):
            for j in range(nsub):
                s = pl.ds(j * L, L)
                rows_v[e, s] = rows_v[e, s] * w_v[e, s]
            return 0

        lax.fori_loop(0, CH, emul, 0)
        pltpu.sync_copy(rows_v, agg_sh.at[idx_d], add=True)
        return 0

    lax.fori_loop(0, per_tile, chunk_body, 0)
    plsc.subcore_barrier()

    out_base = cid * n_nodes + sid * rows_per
    pltpu.sync_copy(agg_sh.at[pl.ds(base_r, rows_per)],
                    out_hbm.at[pl.ds(out_base, rows_per)])


def kernel(h, edge_index, edge_weight, edge_attr, lin1_w, lin2_w, lin2_b,
           mlp_w1, mlp_b1, mlp_w2, mlp_b2):
    n, d = h.shape
    e = edge_weight.shape[0]
    ng = edge_attr.shape[1]
    nf = mlp_w1.shape[0]

    chunk_all = NW * CH
    e_pad = ((e + chunk_all - 1) // chunk_all) * chunk_all
    per_tile = e_pad // chunk_all

    dst = jnp.pad(edge_index[0], (0, e_pad - e))
    src = jnp.pad(edge_index[1], (0, e_pad - e))
    ew = jnp.pad(edge_weight, (0, e_pad - e)).reshape(e_pad, 1)
    attr = jnp.pad(edge_attr, ((0, e_pad - e), (0, 0)))
    b1 = mlp_b1.reshape(1, nf)
    b2 = mlp_b2.reshape(1, nf)
    bo = lin2_b.reshape(1, d)

    rb = 2000
    h1 = pl.pallas_call(
        _lin_body,
        grid=(n // rb,),
        in_specs=[pl.BlockSpec((rb, d), lambda i: (i, 0)),
                  pl.BlockSpec((nf, d), lambda i: (0, 0))],
        out_specs=pl.BlockSpec((rb, nf), lambda i: (i, 0)),
        out_shape=jax.ShapeDtypeStruct((n, nf), jnp.float32),
    )(h, lin1_w)

    eb = 2048
    w_edges = pl.pallas_call(
        functools.partial(_filter_body, e, eb),
        grid=(e_pad // eb,),
        in_specs=[pl.BlockSpec((eb, ng), lambda i: (i, 0)),
                  pl.BlockSpec((eb, 1), lambda i: (i, 0)),
                  pl.BlockSpec((nf, ng), lambda i: (0, 0)),
                  pl.BlockSpec((1, nf), lambda i: (0, 0)),
                  pl.BlockSpec((nf, nf), lambda i: (0, 0)),
                  pl.BlockSpec((1, nf), lambda i: (0, 0))],
        out_specs=pl.BlockSpec((eb, nf), lambda i: (i, 0)),
        out_shape=jax.ShapeDtypeStruct((e_pad, nf), jnp.float32),
    )(attr, ew, mlp_w1, b1, mlp_w2, b2)

    mesh = plsc.VectorSubcoreMesh(core_axis_name="c", subcore_axis_name="s")
    partials = pl.kernel(
        functools.partial(_sc_gather_mul_scatter, n, d, per_tile),
        out_type=jax.ShapeDtypeStruct((NC * n, d), jnp.float32),
        mesh=mesh,
        scratch_types=[
            pltpu.VMEM((CH,), jnp.int32),
            pltpu.VMEM((CH,), jnp.int32),
            pltpu.VMEM((CH, d), jnp.float32),
            pltpu.VMEM((CH, d), jnp.float32),
            pltpu.VMEM_SHARED((n, d), jnp.float32),
            pltpu.SemaphoreType.DMA,
        ],
    )(h1, src, dst, w_edges)

    nb = n // rb
    out = pl.pallas_call(
        _out_body,
        grid=(nb,),
        in_specs=[pl.BlockSpec((rb, nf), lambda i: (i, 0)),
                  pl.BlockSpec((rb, nf), lambda i: (i, 0)),
                  pl.BlockSpec((rb, nf), lambda i: (i + nb, 0)),
                  pl.BlockSpec((d, nf), lambda i: (0, 0)),
                  pl.BlockSpec((1, d), lambda i: (0, 0))],
        out_specs=pl.BlockSpec((rb, d), lambda i: (i, 0)),
        out_shape=jax.ShapeDtypeStruct((n, d), jnp.float32),
    )(h1, partials, partials, lin2_w, bo)

    return out


# speedup vs baseline: 1.3777x; 1.3777x over previous
import functools

import jax
import jax.numpy as jnp
import numpy as np
from jax import lax
from jax.experimental import pallas as pl
from jax.experimental.pallas import tpu as pltpu
from jax.experimental.pallas import tpu_sc as plsc

CUTOFF = 10.0
LOG2 = float(np.log(2.0))
PI = float(np.pi)

NC = 2
NS = 16
L = 16
NW = NC * NS
CH = 128


def _lin_body(x_ref, w_ref, o_ref):
    o_ref[...] = lax.dot_general(
        x_ref[...], w_ref[...], (((1,), (1,)), ((), ())),
        preferred_element_type=jnp.float32)


def _filter_body(n_edges, eb, attr_ref, ew_ref, w1_ref, b1_ref, w2_ref, b2_ref,
                 o_ref):
    x = lax.dot_general(attr_ref[...], w1_ref[...], (((1,), (1,)), ((), ())),
                        preferred_element_type=jnp.float32)
    x = x + b1_ref[...]
    x = jnp.maximum(x, 0.0) + jnp.log(1.0 + jnp.exp(-jnp.abs(x))) - LOG2
    x = lax.dot_general(x, w2_ref[...], (((1,), (1,)), ((), ())),
                        preferred_element_type=jnp.float32)
    x = x + b2_ref[...]
    ew = ew_ref[...]
    c = 0.5 * (jnp.cos(ew * (PI / CUTOFF)) + 1.0)
    rows = pl.program_id(0) * eb + lax.broadcasted_iota(jnp.int32, ew.shape, 0)
    c = jnp.where(rows < n_edges, c, 0.0)
    o_ref[...] = x * c


def _out_body(h1_ref, a0_ref, a1_ref, w_ref, b_ref, o_ref):
    s = h1_ref[...] + a0_ref[...] + a1_ref[...]
    o_ref[...] = lax.dot_general(
        s, w_ref[...], (((1,), (1,)), ((), ())),
        preferred_element_type=jnp.float32) + b_ref[...]


def _sc_gather_mul_scatter(n_nodes, d, per_tile,
                           h1_hbm, src_hbm, dst_hbm, w_hbm, out_hbm,
                           idx_s, idx_d, rows_v, w_v, agg_sh, gsem):
    cid = lax.axis_index("c")
    sid = lax.axis_index("s")
    wid = cid * NS + sid
    nsub = d // L
    rows_lo = (n_nodes // NS) // 8 * 8
    tail = n_nodes - NS * rows_lo

    zero = jnp.zeros((L,), jnp.float32)

    def zb(i, _):
        w_v[i // nsub, pl.ds((i % nsub) * L, L)] = zero
        return 0

    lax.fori_loop(0, CH * nsub, zb, 0)
    base_r = pl.multiple_of(sid * rows_lo, 8)
    nfull = rows_lo // CH
    rem = rows_lo - nfull * CH
    for k in range(nfull):
        pltpu.sync_copy(w_v, agg_sh.at[pl.ds(base_r + k * CH, CH)])
    if rem:
        pltpu.sync_copy(w_v.at[pl.ds(0, rem)],
                        agg_sh.at[pl.ds(base_r + nfull * CH, rem)])
    if tail:
        @pl.when(sid == NS - 1)
        def _zero_tail():
            pltpu.sync_copy(w_v.at[pl.ds(0, tail)],
                            agg_sh.at[pl.ds(NS * rows_lo, tail)])
    plsc.subcore_barrier()

    def chunk_body(ci, _):
        base = (wid * per_tile + ci) * CH
        pltpu.sync_copy(src_hbm.at[pl.ds(base, CH)], idx_s)
        pltpu.sync_copy(dst_hbm.at[pl.ds(base, CH)], idx_d)
        pltpu.async_copy(h1_hbm.at[idx_s], rows_v, gsem).wait()
        pltpu.sync_copy(w_hbm.at[pl.ds(base, CH)], w_v)

        def emul(e, _):
            for j in range(nsub):
                s = pl.ds(j * L, L)
                rows_v[e, s] = rows_v[e, s] * w_v[e, s]
            return 0

        lax.fori_loop(0, CH, emul, 0)
        pltpu.sync_copy(rows_v, agg_sh.at[idx_d], add=True)
        return 0

    lax.fori_loop(0, per_tile, chunk_body, 0)
    plsc.subcore_barrier()

    out_base = pl.multiple_of(cid * n_nodes + sid * rows_lo, 8)
    pltpu.sync_copy(agg_sh.at[pl.ds(base_r, rows_lo)],
                    out_hbm.at[pl.ds(out_base, rows_lo)])
    if tail:
        @pl.when(sid == NS - 1)
        def _write_tail():
            tb = pl.multiple_of(cid * n_nodes + NS * rows_lo, 8)
            pltpu.sync_copy(agg_sh.at[pl.ds(NS * rows_lo, tail)],
                            out_hbm.at[pl.ds(tb, tail)])


def kernel(h, edge_index, edge_weight, edge_attr, lin1_w, lin2_w, lin2_b,
           mlp_w1, mlp_b1, mlp_w2, mlp_b2):
    n, d = h.shape
    e = edge_weight.shape[0]
    ng = edge_attr.shape[1]
    nf = mlp_w1.shape[0]

    chunk_all = NW * CH
    e_pad = ((e + chunk_all - 1) // chunk_all) * chunk_all
    per_tile = e_pad // chunk_all

    dst = jnp.pad(edge_index[0], (0, e_pad - e))
    src = jnp.pad(edge_index[1], (0, e_pad - e))
    ew = jnp.pad(edge_weight, (0, e_pad - e)).reshape(e_pad, 1)
    attr = jnp.pad(edge_attr, ((0, e_pad - e), (0, 0)))
    b1 = mlp_b1.reshape(1, nf)
    b2 = mlp_b2.reshape(1, nf)
    bo = lin2_b.reshape(1, d)

    rb = 2000
    h1 = pl.pallas_call(
        _lin_body,
        grid=(n // rb,),
        in_specs=[pl.BlockSpec((rb, d), lambda i: (i, 0)),
                  pl.BlockSpec((nf, d), lambda i: (0, 0))],
        out_specs=pl.BlockSpec((rb, nf), lambda i: (i, 0)),
        out_shape=jax.ShapeDtypeStruct((n, nf), jnp.float32),
    )(h, lin1_w)

    eb = 2048
    w_edges = pl.pallas_call(
        functools.partial(_filter_body, e, eb),
        grid=(e_pad // eb,),
        in_specs=[pl.BlockSpec((eb, ng), lambda i: (i, 0)),
                  pl.BlockSpec((eb, 1), lambda i: (i, 0)),
                  pl.BlockSpec((nf, ng), lambda i: (0, 0)),
                  pl.BlockSpec((1, nf), lambda i: (0, 0)),
                  pl.BlockSpec((nf, nf), lambda i: (0, 0)),
                  pl.BlockSpec((1, nf), lambda i: (0, 0))],
        out_specs=pl.BlockSpec((eb, nf), lambda i: (i, 0)),
        out_shape=jax.ShapeDtypeStruct((e_pad, nf), jnp.float32),
    )(attr, ew, mlp_w1, b1, mlp_w2, b2)

    mesh = plsc.VectorSubcoreMesh(core_axis_name="c", subcore_axis_name="s")
    partials = pl.kernel(
        functools.partial(_sc_gather_mul_scatter, n, d, per_tile),
        out_type=jax.ShapeDtypeStruct((NC * n, d), jnp.float32),
        mesh=mesh,
        scratch_types=[
            pltpu.VMEM((CH,), jnp.int32),
            pltpu.VMEM((CH,), jnp.int32),
            pltpu.VMEM((CH, d), jnp.float32),
            pltpu.VMEM((CH, d), jnp.float32),
            pltpu.VMEM_SHARED((n, d), jnp.float32),
            pltpu.SemaphoreType.DMA,
        ],
    )(h1, src, dst, w_edges)

    nb = n // rb
    out = pl.pallas_call(
        _out_body,
        grid=(nb,),
        in_specs=[pl.BlockSpec((rb, nf), lambda i: (i, 0)),
                  pl.BlockSpec((rb, nf), lambda i: (i, 0)),
                  pl.BlockSpec((rb, nf), lambda i: (i + nb, 0)),
                  pl.BlockSpec((d, nf), lambda i: (0, 0)),
                  pl.BlockSpec((1, d), lambda i: (0, 0))],
        out_specs=pl.BlockSpec((rb, d), lambda i: (i, 0)),
        out_shape=jax.ShapeDtypeStruct((n, d), jnp.float32),
    )(h1, partials, partials, lin2_w, bo)

    return out


# speedup vs baseline: 1.5472x; 1.1231x over previous
import functools

import jax
import jax.numpy as jnp
import numpy as np
from jax import lax
from jax.experimental import pallas as pl
from jax.experimental.pallas import tpu as pltpu
from jax.experimental.pallas import tpu_sc as plsc

CUTOFF = 10.0
LOG2 = float(np.log(2.0))
PI = float(np.pi)

NC = 2
NS = 16
L = 16
NW = NC * NS
CH = 96


def _lin_body(x_ref, w_ref, o_ref):
    o_ref[...] = lax.dot_general(
        x_ref[...], w_ref[...], (((1,), (1,)), ((), ())),
        preferred_element_type=jnp.float32)


def _filter_body(n_edges, eb, attr_ref, ew_ref, w1_ref, b1_ref, w2_ref, b2_ref,
                 o_ref):
    x = lax.dot_general(attr_ref[...], w1_ref[...], (((1,), (1,)), ((), ())),
                        preferred_element_type=jnp.float32)
    x = x + b1_ref[...]
    x = jnp.maximum(x, 0.0) + jnp.log(1.0 + jnp.exp(-jnp.abs(x))) - LOG2
    x = lax.dot_general(x, w2_ref[...], (((1,), (1,)), ((), ())),
                        preferred_element_type=jnp.float32)
    x = x + b2_ref[...]
    ew = ew_ref[...]
    c = 0.5 * (jnp.cos(ew * (PI / CUTOFF)) + 1.0)
    rows = pl.program_id(0) * eb + lax.broadcasted_iota(jnp.int32, ew.shape, 0)
    c = jnp.where(rows < n_edges, c, 0.0)
    o_ref[...] = x * c


def _out_body(h1_ref, a0_ref, a1_ref, w_ref, b_ref, o_ref):
    s = h1_ref[...] + a0_ref[...] + a1_ref[...]
    o_ref[...] = lax.dot_general(
        s, w_ref[...], (((1,), (1,)), ((), ())),
        preferred_element_type=jnp.float32) + b_ref[...]


def _sc_gather_mul_scatter(n_nodes, d, per_tile,
                           h1_hbm, idx2_hbm, w_hbm, out_hbm,
                           idx_v, rows_v, w_v, agg_sh,
                           isem0, isem1, gsem0, gsem1, wsem0, wsem1):
    cid = lax.axis_index("c")
    sid = lax.axis_index("s")
    wid = cid * NS + sid
    nsub = d // L
    isem = (isem0, isem1)
    gsem = (gsem0, gsem1)
    wsem = (wsem0, wsem1)
    rows_lo = (n_nodes // NS) // 8 * 8
    tail = n_nodes - NS * rows_lo

    zero = jnp.zeros((L,), jnp.float32)

    def zb(i, _):
        w_v[0, i // nsub, pl.ds((i % nsub) * L, L)] = zero
        return 0

    lax.fori_loop(0, CH * nsub, zb, 0)
    base_r = pl.multiple_of(sid * rows_lo, 8)
    nfull = rows_lo // CH
    rem = rows_lo - nfull * CH
    for k in range(nfull):
        pltpu.sync_copy(w_v.at[0], agg_sh.at[pl.ds(base_r + k * CH, CH)])
    if rem:
        pltpu.sync_copy(w_v.at[0, pl.ds(0, rem)],
                        agg_sh.at[pl.ds(base_r + nfull * CH, rem)])
    if tail:
        @pl.when(sid == NS - 1)
        def _zero_tail():
            pltpu.sync_copy(w_v.at[0, pl.ds(0, tail)],
                            agg_sh.at[pl.ds(NS * rows_lo, tail)])
    plsc.subcore_barrier()

    def idx_desc(ci, p):
        return pltpu.make_async_copy(
            idx2_hbm.at[pl.ds((wid * per_tile + ci) * 2, 2)],
            idx_v.at[pl.ds(2 * p, 2)], isem[p])

    def g_desc(p):
        return pltpu.make_async_copy(
            h1_hbm.at[idx_v.at[2 * p]], rows_v.at[p], gsem[p])

    def w_desc(ci, p):
        return pltpu.make_async_copy(
            w_hbm.at[pl.ds((wid * per_tile + ci) * CH, CH)],
            w_v.at[p], wsem[p])

    idx_desc(0, 0).start()
    idx_desc(1, 1).start()
    idx_desc(0, 0).wait()
    g_desc(0).start()
    w_desc(0, 0).start()

    def pair_body(k, _):
        for p in (0, 1):
            ci = 2 * k + p
            q = 1 - p

            @pl.when(ci + 1 < per_tile)
            def _prefetch_gw():
                idx_desc(ci + 1, q).wait()
                g_desc(q).start()
                w_desc(ci + 1, q).start()

            g_desc(p).wait()
            w_desc(ci, p).wait()
            rvp = rows_v.at[p]
            wvp = w_v.at[p]

            @plsc.parallel_loop(0, CH, step=1, unroll=4)
            def _mul(ei):
                for j in range(nsub):
                    s = pl.ds(j * L, L)
                    rvp[ei, s] = rvp[ei, s] * wvp[ei, s]

            pltpu.sync_copy(rvp, agg_sh.at[idx_v.at[2 * p + 1]], add=True)

            @pl.when(ci + 2 < per_tile)
            def _prefetch_idx():
                idx_desc(ci + 2, p).start()
        return 0

    lax.fori_loop(0, per_tile // 2, pair_body, 0)
    plsc.subcore_barrier()

    out_base = pl.multiple_of(cid * n_nodes + sid * rows_lo, 8)
    pltpu.sync_copy(agg_sh.at[pl.ds(base_r, rows_lo)],
                    out_hbm.at[pl.ds(out_base, rows_lo)])
    if tail:
        @pl.when(sid == NS - 1)
        def _write_tail():
            tb = pl.multiple_of(cid * n_nodes + NS * rows_lo, 8)
            pltpu.sync_copy(agg_sh.at[pl.ds(NS * rows_lo, tail)],
                            out_hbm.at[pl.ds(tb, tail)])


def kernel(h, edge_index, edge_weight, edge_attr, lin1_w, lin2_w, lin2_b,
           mlp_w1, mlp_b1, mlp_w2, mlp_b2):
    n, d = h.shape
    e = edge_weight.shape[0]
    ng = edge_attr.shape[1]
    nf = mlp_w1.shape[0]

    chunk_all = NW * CH * 2
    e_pad = ((e + chunk_all - 1) // chunk_all) * chunk_all
    per_tile = e_pad // (NW * CH)

    dst = jnp.pad(edge_index[0], (0, e_pad - e))
    src = jnp.pad(edge_index[1], (0, e_pad - e))
    idx2 = jnp.stack([src.reshape(e_pad // CH, CH),
                      dst.reshape(e_pad // CH, CH)], axis=1).reshape(-1, CH)
    ew = jnp.pad(edge_weight, (0, e_pad - e)).reshape(e_pad, 1)
    attr = jnp.pad(edge_attr, ((0, e_pad - e), (0, 0)))
    b1 = mlp_b1.reshape(1, nf)
    b2 = mlp_b2.reshape(1, nf)
    bo = lin2_b.reshape(1, d)

    rb = 2000
    h1 = pl.pallas_call(
        _lin_body,
        grid=(n // rb,),
        in_specs=[pl.BlockSpec((rb, d), lambda i: (i, 0)),
                  pl.BlockSpec((nf, d), lambda i: (0, 0))],
        out_specs=pl.BlockSpec((rb, nf), lambda i: (i, 0)),
        out_shape=jax.ShapeDtypeStruct((n, nf), jnp.float32),
    )(h, lin1_w)

    eb = 2048
    w_edges = pl.pallas_call(
        functools.partial(_filter_body, e, eb),
        grid=(e_pad // eb,),
        in_specs=[pl.BlockSpec((eb, ng), lambda i: (i, 0)),
                  pl.BlockSpec((eb, 1), lambda i: (i, 0)),
                  pl.BlockSpec((nf, ng), lambda i: (0, 0)),
                  pl.BlockSpec((1, nf), lambda i: (0, 0)),
                  pl.BlockSpec((nf, nf), lambda i: (0, 0)),
                  pl.BlockSpec((1, nf), lambda i: (0, 0))],
        out_specs=pl.BlockSpec((eb, nf), lambda i: (i, 0)),
        out_shape=jax.ShapeDtypeStruct((e_pad, nf), jnp.float32),
    )(attr, ew, mlp_w1, b1, mlp_w2, b2)

    mesh = plsc.VectorSubcoreMesh(core_axis_name="c", subcore_axis_name="s")
    partials = pl.kernel(
        functools.partial(_sc_gather_mul_scatter, n, d, per_tile),
        out_type=jax.ShapeDtypeStruct((NC * n, d), jnp.float32),
        mesh=mesh,
        scratch_types=[
            pltpu.VMEM((4, CH), jnp.int32),
            pltpu.VMEM((2, CH, d), jnp.float32),
            pltpu.VMEM((2, CH, d), jnp.float32),
            pltpu.VMEM_SHARED((n, d), jnp.float32),
            pltpu.SemaphoreType.DMA,
            pltpu.SemaphoreType.DMA,
            pltpu.SemaphoreType.DMA,
            pltpu.SemaphoreType.DMA,
            pltpu.SemaphoreType.DMA,
            pltpu.SemaphoreType.DMA,
        ],
    )(h1, idx2, w_edges)

    nb = n // rb
    out = pl.pallas_call(
        _out_body,
        grid=(nb,),
        in_specs=[pl.BlockSpec((rb, nf), lambda i: (i, 0)),
                  pl.BlockSpec((rb, nf), lambda i: (i, 0)),
                  pl.BlockSpec((rb, nf), lambda i: (i + nb, 0)),
                  pl.BlockSpec((d, nf), lambda i: (0, 0)),
                  pl.BlockSpec((1, d), lambda i: (0, 0))],
        out_specs=pl.BlockSpec((rb, d), lambda i: (i, 0)),
        out_shape=jax.ShapeDtypeStruct((n, d), jnp.float32),
    )(h1, partials, partials, lin2_w, bo)

    return out


# speedup vs baseline: 1.8512x; 1.1964x over previous
import functools

import jax
import jax.numpy as jnp
import numpy as np
from jax import lax
from jax.experimental import pallas as pl
from jax.experimental.pallas import tpu as pltpu
from jax.experimental.pallas import tpu_sc as plsc

CUTOFF = 10.0
LOG2 = float(np.log(2.0))
PI = float(np.pi)

NC = 2
NS = 16
L = 16
NW = NC * NS
CH = 80


def _lin_body(x_ref, w_ref, o_ref):
    o_ref[...] = lax.dot_general(
        x_ref[...], w_ref[...], (((1,), (1,)), ((), ())),
        preferred_element_type=jnp.float32)


def _filter_body(attr_ref, ew_ref, w1_ref, b1_ref, w2_ref, b2_ref, o_ref):
    x = lax.dot_general(attr_ref[...], w1_ref[...], (((1,), (1,)), ((), ())),
                        preferred_element_type=jnp.float32)
    x = x + b1_ref[...]
    x = jnp.maximum(x, 0.0) + jnp.log(1.0 + jnp.exp(-jnp.abs(x))) - LOG2
    x = lax.dot_general(x, w2_ref[...], (((1,), (1,)), ((), ())),
                        preferred_element_type=jnp.float32)
    x = x + b2_ref[...]
    c = 0.5 * (jnp.cos(ew_ref[...] * (PI / CUTOFF)) + 1.0)
    o_ref[...] = x * c


def _out_body(h1_ref, a0_ref, a1_ref, w_ref, b_ref, o_ref):
    s = h1_ref[...] + a0_ref[...] + a1_ref[...]
    o_ref[...] = lax.dot_general(
        s, w_ref[...], (((1,), (1,)), ((), ())),
        preferred_element_type=jnp.float32) + b_ref[...]


def _sc_gather_mul_scatter(n_nodes, d, per_tile,
                           h1_hbm, src_hbm, dst_hbm, w_hbm, out_hbm,
                           src_v, dst_v, rows_v, w_v, agg_sh,
                           ssem0, ssem1, dsem0, dsem1,
                           gsem0, gsem1, wsem0, wsem1):
    cid = lax.axis_index("c")
    sid = lax.axis_index("s")
    wid = cid * NS + sid
    nsub = d // L
    ssem = (ssem0, ssem1)
    dsem = (dsem0, dsem1)
    gsem = (gsem0, gsem1)
    wsem = (wsem0, wsem1)
    rows_lo = (n_nodes // NS) // 8 * 8
    tail = n_nodes - NS * rows_lo

    zero = jnp.zeros((L,), jnp.float32)

    def zb(i, _):
        w_v[0, i // nsub, pl.ds((i % nsub) * L, L)] = zero
        return 0

    lax.fori_loop(0, CH * nsub, zb, 0)
    base_r = pl.multiple_of(sid * rows_lo, 8)
    nfull = rows_lo // CH
    rem = rows_lo - nfull * CH
    for k in range(nfull):
        pltpu.sync_copy(w_v.at[0], agg_sh.at[pl.ds(base_r + k * CH, CH)])
    if rem:
        pltpu.sync_copy(w_v.at[0, pl.ds(0, rem)],
                        agg_sh.at[pl.ds(base_r + nfull * CH, rem)])
    if tail:
        @pl.when(sid == NS - 1)
        def _zero_tail():
            pltpu.sync_copy(w_v.at[0, pl.ds(0, tail)],
                            agg_sh.at[pl.ds(NS * rows_lo, tail)])
    plsc.subcore_barrier()

    def sidx_desc(ci, p):
        base = pl.multiple_of((wid * per_tile + ci) * CH, 8)
        return pltpu.make_async_copy(
            src_hbm.at[pl.ds(base, CH)], src_v.at[p], ssem[p])

    def didx_desc(ci, p):
        base = pl.multiple_of((wid * per_tile + ci) * CH, 8)
        return pltpu.make_async_copy(
            dst_hbm.at[pl.ds(base, CH)], dst_v.at[p], dsem[p])

    def g_desc(p):
        return pltpu.make_async_copy(
            h1_hbm.at[src_v.at[p]], rows_v.at[p], gsem[p])

    def w_desc(ci, p):
        base = pl.multiple_of((wid * per_tile + ci) * CH, 8)
        return pltpu.make_async_copy(
            w_hbm.at[pl.ds(base, CH)], w_v.at[p], wsem[p])

    sidx_desc(0, 0).start()
    didx_desc(0, 0).start()
    sidx_desc(1, 1).start()
    didx_desc(1, 1).start()
    sidx_desc(0, 0).wait()
    g_desc(0).start()
    w_desc(0, 0).start()

    def step(ci, p):
        q = 1 - p

        @pl.when(ci + 1 < per_tile)
        def _prefetch_gw():
            sidx_desc(ci + 1, q).wait()
            g_desc(q).start()
            w_desc(ci + 1, q).start()

        g_desc(p).wait()
        w_desc(ci, p).wait()
        rvp = rows_v.at[p]
        wvp = w_v.at[p]

        @plsc.parallel_loop(0, CH, step=1, unroll=4)
        def _mul(ei):
            for j in range(nsub):
                s = pl.ds(j * L, L)
                rvp[ei, s] = rvp[ei, s] * wvp[ei, s]

        didx_desc(ci, p).wait()
        pltpu.sync_copy(rvp, agg_sh.at[dst_v.at[p]], add=True)

        @pl.when(ci + 2 < per_tile)
        def _prefetch_idx():
            sidx_desc(ci + 2, p).start()
            didx_desc(ci + 2, p).start()

    def pair_body(k, _):
        step(2 * k, 0)
        step(2 * k + 1, 1)
        return 0

    lax.fori_loop(0, per_tile // 2, pair_body, 0)
    if per_tile % 2:
        step(per_tile - 1, (per_tile - 1) % 2)
    plsc.subcore_barrier()

    out_base = pl.multiple_of(cid * n_nodes + sid * rows_lo, 8)
    pltpu.sync_copy(agg_sh.at[pl.ds(base_r, rows_lo)],
                    out_hbm.at[pl.ds(out_base, rows_lo)])
    if tail:
        @pl.when(sid == NS - 1)
        def _write_tail():
            tb = pl.multiple_of(cid * n_nodes + NS * rows_lo, 8)
            pltpu.sync_copy(agg_sh.at[pl.ds(NS * rows_lo, tail)],
                            out_hbm.at[pl.ds(tb, tail)])


def kernel(h, edge_index, edge_weight, edge_attr, lin1_w, lin2_w, lin2_b,
           mlp_w1, mlp_b1, mlp_w2, mlp_b2):
    n, d = h.shape
    e = edge_weight.shape[0]
    ng = edge_attr.shape[1]
    nf = mlp_w1.shape[0]
    assert e % (NW * CH) == 0
    per_tile = e // (NW * CH)

    dst = edge_index[0]
    src = edge_index[1]
    ew = edge_weight.reshape(e, 1)
    b1 = mlp_b1.reshape(1, nf)
    b2 = mlp_b2.reshape(1, nf)
    bo = lin2_b.reshape(1, d)

    rb = 2000
    h1 = pl.pallas_call(
        _lin_body,
        grid=(n // rb,),
        in_specs=[pl.BlockSpec((rb, d), lambda i: (i, 0)),
                  pl.BlockSpec((nf, d), lambda i: (0, 0))],
        out_specs=pl.BlockSpec((rb, nf), lambda i: (i, 0)),
        out_shape=jax.ShapeDtypeStruct((n, nf), jnp.float32),
    )(h, lin1_w)

    eb = 2000
    w_edges = pl.pallas_call(
        _filter_body,
        grid=(e // eb,),
        in_specs=[pl.BlockSpec((eb, ng), lambda i: (i, 0)),
                  pl.BlockSpec((eb, 1), lambda i: (i, 0)),
                  pl.BlockSpec((nf, ng), lambda i: (0, 0)),
                  pl.BlockSpec((1, nf), lambda i: (0, 0)),
                  pl.BlockSpec((nf, nf), lambda i: (0, 0)),
                  pl.BlockSpec((1, nf), lambda i: (0, 0))],
        out_specs=pl.BlockSpec((eb, nf), lambda i: (i, 0)),
        out_shape=jax.ShapeDtypeStruct((e, nf), jnp.float32),
    )(edge_attr, ew, mlp_w1, b1, mlp_w2, b2)

    mesh = plsc.VectorSubcoreMesh(core_axis_name="c", subcore_axis_name="s")
    partials = pl.kernel(
        functools.partial(_sc_gather_mul_scatter, n, d, per_tile),
        out_type=jax.ShapeDtypeStruct((NC * n, d), jnp.float32),
        mesh=mesh,
        scratch_types=[
            pltpu.VMEM((2, CH), jnp.int32),
            pltpu.VMEM((2, CH), jnp.int32),
            pltpu.VMEM((2, CH, d), jnp.float32),
            pltpu.VMEM((2, CH, d), jnp.float32),
            pltpu.VMEM_SHARED((n, d), jnp.float32),
            pltpu.SemaphoreType.DMA,
            pltpu.SemaphoreType.DMA,
            pltpu.SemaphoreType.DMA,
            pltpu.SemaphoreType.DMA,
            pltpu.SemaphoreType.DMA,
            pltpu.SemaphoreType.DMA,
            pltpu.SemaphoreType.DMA,
            pltpu.SemaphoreType.DMA,
        ],
    )(h1, src, dst, w_edges)

    nb = n // rb
    out = pl.pallas_call(
        _out_body,
        grid=(nb,),
        in_specs=[pl.BlockSpec((rb, nf), lambda i: (i, 0)),
                  pl.BlockSpec((rb, nf), lambda i: (i, 0)),
                  pl.BlockSpec((rb, nf), lambda i: (i + nb, 0)),
                  pl.BlockSpec((d, nf), lambda i: (0, 0)),
                  pl.BlockSpec((1, d), lambda i: (0, 0))],
        out_specs=pl.BlockSpec((rb, d), lambda i: (i, 0)),
        out_shape=jax.ShapeDtypeStruct((n, d), jnp.float32),
    )(h1, partials, partials, lin2_w, bo)

    return out


# speedup vs baseline: 3.7818x; 2.0429x over previous
import functools

import jax
import jax.numpy as jnp
import numpy as np
from jax import lax
from jax.experimental import pallas as pl
from jax.experimental.pallas import tpu as pltpu
from jax.experimental.pallas import tpu_sc as plsc

CUTOFF = 10.0
LOG2 = float(np.log(2.0))
PI = float(np.pi)

NC = 2
NS = 16
L = 16
NW = NC * NS
CH = 80


def _lin_body(x_ref, w_ref, o_ref):
    o_ref[...] = lax.dot_general(
        x_ref[...], w_ref[...], (((1,), (1,)), ((), ())),
        preferred_element_type=jnp.float32)


def _filter_body(attr_ref, w1_ref, b1_ref, w2_ref, b2_ref, o_ref):
    x = lax.dot_general(attr_ref[...], w1_ref[...], (((1,), (1,)), ((), ())),
                        preferred_element_type=jnp.float32)
    x = x + b1_ref[...]
    x = jnp.maximum(x, 0.0) + jnp.log(1.0 + jnp.exp(-jnp.abs(x))) - LOG2
    x = lax.dot_general(x, w2_ref[...], (((1,), (1,)), ((), ())),
                        preferred_element_type=jnp.float32)
    o_ref[...] = x + b2_ref[...]


def _cutoff_body(ew_ref, c_ref):
    c_ref[...] = 0.5 * (jnp.cos(ew_ref[...] * (PI / CUTOFF)) + 1.0)


def _out_body(h1_ref, a0_ref, a1_ref, w_ref, b_ref, o_ref):
    s = h1_ref[...] + a0_ref[...] + a1_ref[...]
    o_ref[...] = lax.dot_general(
        s, w_ref[...], (((1,), (1,)), ((), ())),
        preferred_element_type=jnp.float32) + b_ref[...]


def _sc_gather_mul_scatter(n_nodes, d, per_tile, n_edges,
                           h1_hbm, ei_hbm, w_hbm, c_hbm, out_hbm,
                           src_v, dst_v, c_v, rows_v, w_v, agg_sh,
                           ssem0, ssem1, dsem0, dsem1,
                           gsem0, gsem1, wsem0, wsem1, csem0, csem1):
    cid = lax.axis_index("c")
    sid = lax.axis_index("s")
    wid = cid * NS + sid
    nsub = d // L
    ssem = (ssem0, ssem1)
    dsem = (dsem0, dsem1)
    gsem = (gsem0, gsem1)
    wsem = (wsem0, wsem1)
    csem = (csem0, csem1)
    rows_lo = (n_nodes // NS) // 8 * 8
    tail = n_nodes - NS * rows_lo

    zero = jnp.zeros((L,), jnp.float32)

    def zb(i, _):
        w_v[0, i // nsub, pl.ds((i % nsub) * L, L)] = zero
        return 0

    lax.fori_loop(0, CH * nsub, zb, 0)
    base_r = pl.multiple_of(sid * rows_lo, 8)
    nfull = rows_lo // CH
    rem = rows_lo - nfull * CH
    for k in range(nfull):
        pltpu.sync_copy(w_v.at[0], agg_sh.at[pl.ds(base_r + k * CH, CH)])
    if rem:
        pltpu.sync_copy(w_v.at[0, pl.ds(0, rem)],
                        agg_sh.at[pl.ds(base_r + nfull * CH, rem)])
    if tail:
        @pl.when(sid == NS - 1)
        def _zero_tail():
            pltpu.sync_copy(w_v.at[0, pl.ds(0, tail)],
                            agg_sh.at[pl.ds(NS * rows_lo, tail)])
    plsc.subcore_barrier()

    def sidx_desc(ci, p):
        base = pl.multiple_of(n_edges + (wid * per_tile + ci) * CH, 8)
        return pltpu.make_async_copy(
            ei_hbm.at[pl.ds(base, CH)], src_v.at[p], ssem[p])

    def didx_desc(ci, p):
        base = pl.multiple_of((wid * per_tile + ci) * CH, 8)
        return pltpu.make_async_copy(
            ei_hbm.at[pl.ds(base, CH)], dst_v.at[p], dsem[p])

    def g_desc(p):
        return pltpu.make_async_copy(
            h1_hbm.at[src_v.at[p]], rows_v.at[p], gsem[p])

    def w_desc(ci, p):
        base = pl.multiple_of((wid * per_tile + ci) * CH, 8)
        return pltpu.make_async_copy(
            w_hbm.at[pl.ds(base, CH)], w_v.at[p], wsem[p])

    def c_desc(ci, p):
        base = pl.multiple_of((wid * per_tile + ci) * CH, 8)
        return pltpu.make_async_copy(
            c_hbm.at[pl.ds(base, CH)], c_v.at[p, pl.ds(0, CH)], csem[p])

    sidx_desc(0, 0).start()
    didx_desc(0, 0).start()
    sidx_desc(1, 1).start()
    didx_desc(1, 1).start()
    sidx_desc(0, 0).wait()
    g_desc(0).start()
    w_desc(0, 0).start()
    c_desc(0, 0).start()

    def step(ci, p):
        q = 1 - p

        @pl.when(ci + 1 < per_tile)
        def _prefetch_gw():
            sidx_desc(ci + 1, q).wait()
            g_desc(q).start()
            w_desc(ci + 1, q).start()
            c_desc(ci + 1, q).start()

        g_desc(p).wait()
        w_desc(ci, p).wait()
        c_desc(ci, p).wait()
        rvp = rows_v.at[p]
        wvp = w_v.at[p]
        cvp = c_v.at[p]

        @plsc.parallel_loop(0, CH, step=1, unroll=4)
        def _mul(ei):
            cs = cvp[pl.ds(ei, L)][0]
            for j in range(nsub):
                s = pl.ds(j * L, L)
                rvp[ei, s] = rvp[ei, s] * (wvp[ei, s] * cs)

        didx_desc(ci, p).wait()
        pltpu.sync_copy(rvp, agg_sh.at[dst_v.at[p]], add=True)

        @pl.when(ci + 2 < per_tile)
        def _prefetch_idx():
            sidx_desc(ci + 2, p).start()
            didx_desc(ci + 2, p).start()

    def pair_body(k, _):
        step(2 * k, 0)
        step(2 * k + 1, 1)
        return 0

    lax.fori_loop(0, per_tile // 2, pair_body, 0)
    if per_tile % 2:
        step(per_tile - 1, (per_tile - 1) % 2)
    plsc.subcore_barrier()

    out_base = pl.multiple_of(cid * n_nodes + sid * rows_lo, 8)
    pltpu.sync_copy(agg_sh.at[pl.ds(base_r, rows_lo)],
                    out_hbm.at[pl.ds(out_base, rows_lo)])
    if tail:
        @pl.when(sid == NS - 1)
        def _write_tail():
            tb = pl.multiple_of(cid * n_nodes + NS * rows_lo, 8)
            pltpu.sync_copy(agg_sh.at[pl.ds(NS * rows_lo, tail)],
                            out_hbm.at[pl.ds(tb, tail)])


def kernel(h, edge_index, edge_weight, edge_attr, lin1_w, lin2_w, lin2_b,
           mlp_w1, mlp_b1, mlp_w2, mlp_b2):
    n, d = h.shape
    e = edge_weight.shape[0]
    ng = edge_attr.shape[1]
    nf = mlp_w1.shape[0]
    assert e % (NW * CH) == 0
    per_tile = e // (NW * CH)

    ei_flat = edge_index.reshape(2 * e)
    b1 = mlp_b1.reshape(1, nf)
    b2 = mlp_b2.reshape(1, nf)
    bo = lin2_b.reshape(1, d)

    rb = 2000
    h1 = pl.pallas_call(
        _lin_body,
        grid=(n // rb,),
        in_specs=[pl.BlockSpec((rb, d), lambda i: (i, 0)),
                  pl.BlockSpec((nf, d), lambda i: (0, 0))],
        out_specs=pl.BlockSpec((rb, nf), lambda i: (i, 0)),
        out_shape=jax.ShapeDtypeStruct((n, nf), jnp.float32),
    )(h, lin1_w)

    ewv = edge_weight.reshape(e // 128, 128)
    c_edges = pl.pallas_call(
        _cutoff_body,
        out_shape=jax.ShapeDtypeStruct((e // 128, 128), jnp.float32),
    )(ewv)
    c_flat = c_edges.reshape(e)

    eb = 2560
    w_edges = pl.pallas_call(
        _filter_body,
        grid=(e // eb,),
        in_specs=[pl.BlockSpec((eb, ng), lambda i: (i, 0)),
                  pl.BlockSpec((nf, ng), lambda i: (0, 0)),
                  pl.BlockSpec((1, nf), lambda i: (0, 0)),
                  pl.BlockSpec((nf, nf), lambda i: (0, 0)),
                  pl.BlockSpec((1, nf), lambda i: (0, 0))],
        out_specs=pl.BlockSpec((eb, nf), lambda i: (i, 0)),
        out_shape=jax.ShapeDtypeStruct((e, nf), jnp.float32),
    )(edge_attr, mlp_w1, b1, mlp_w2, b2)

    mesh = plsc.VectorSubcoreMesh(core_axis_name="c", subcore_axis_name="s")
    partials = pl.kernel(
        functools.partial(_sc_gather_mul_scatter, n, d, per_tile, e),
        out_type=jax.ShapeDtypeStruct((NC * n, d), jnp.float32),
        mesh=mesh,
        scratch_types=[
            pltpu.VMEM((2, CH), jnp.int32),
            pltpu.VMEM((2, CH), jnp.int32),
            pltpu.VMEM((2, CH + L), jnp.float32),
            pltpu.VMEM((2, CH, d), jnp.float32),
            pltpu.VMEM((2, CH, d), jnp.float32),
            pltpu.VMEM_SHARED((n, d), jnp.float32),
            pltpu.SemaphoreType.DMA,
            pltpu.SemaphoreType.DMA,
            pltpu.SemaphoreType.DMA,
            pltpu.SemaphoreType.DMA,
            pltpu.SemaphoreType.DMA,
            pltpu.SemaphoreType.DMA,
            pltpu.SemaphoreType.DMA,
            pltpu.SemaphoreType.DMA,
            pltpu.SemaphoreType.DMA,
            pltpu.SemaphoreType.DMA,
        ],
    )(h1, ei_flat, w_edges, c_flat)

    nb = n // rb
    out = pl.pallas_call(
        _out_body,
        grid=(nb,),
        in_specs=[pl.BlockSpec((rb, nf), lambda i: (i, 0)),
                  pl.BlockSpec((rb, nf), lambda i: (i, 0)),
                  pl.BlockSpec((rb, nf), lambda i: (i + nb, 0)),
                  pl.BlockSpec((d, nf), lambda i: (0, 0)),
                  pl.BlockSpec((1, d), lambda i: (0, 0))],
        out_specs=pl.BlockSpec((rb, d), lambda i: (i, 0)),
        out_shape=jax.ShapeDtypeStruct((n, d), jnp.float32),
    )(h1, partials, partials, lin2_w, bo)

    return out


# speedup vs baseline: 3.8910x; 1.0289x over previous
import functools

import jax
import jax.numpy as jnp
import numpy as np
from jax import lax
from jax.experimental import pallas as pl
from jax.experimental.pallas import tpu as pltpu
from jax.experimental.pallas import tpu_sc as plsc

CUTOFF = 10.0
LOG2 = float(np.log(2.0))
PI = float(np.pi)

NC = 2
NS = 16
L = 16
NW = NC * NS
CH = 80


def _lin_body(x_ref, w_ref, o_ref):
    o_ref[...] = lax.dot_general(
        x_ref[...], w_ref[...], (((1,), (1,)), ((), ())),
        preferred_element_type=jnp.float32)


def _filter_body(attr_ref, w1_ref, b1_ref, w2_ref, b2_ref, o_ref):
    x = lax.dot_general(attr_ref[...], w1_ref[...], (((1,), (1,)), ((), ())),
                        preferred_element_type=jnp.float32)
    x = x + b1_ref[...]
    x = jnp.maximum(x, 0.0) + jnp.log(1.0 + jnp.exp(-jnp.abs(x))) - LOG2
    x = lax.dot_general(x.astype(jnp.bfloat16), w2_ref[...],
                        (((1,), (1,)), ((), ())),
                        preferred_element_type=jnp.float32)
    o_ref[...] = x + b2_ref[...]


def _cutoff_body(ew_ref, c_ref):
    c_ref[...] = 0.5 * (jnp.cos(ew_ref[...] * (PI / CUTOFF)) + 1.0)


def _out_body(h1_ref, a0_ref, a1_ref, w_ref, b_ref, o_ref):
    s = h1_ref[...] + a0_ref[...] + a1_ref[...]
    o_ref[...] = lax.dot_general(
        s, w_ref[...], (((1,), (1,)), ((), ())),
        preferred_element_type=jnp.float32) + b_ref[...]


def _sc_gather_mul_scatter(n_nodes, d, per_tile, n_edges,
                           h1_hbm, ei_hbm, w_hbm, c_hbm, out_hbm,
                           src_v, dst_v, c_v, rows_v, w_v, agg_sh,
                           ssem0, ssem1, dsem0, dsem1,
                           gsem0, gsem1, wsem0, wsem1, csem0, csem1):
    cid = lax.axis_index("c")
    sid = lax.axis_index("s")
    wid = cid * NS + sid
    nsub = d // L
    ssem = (ssem0, ssem1)
    dsem = (dsem0, dsem1)
    gsem = (gsem0, gsem1)
    wsem = (wsem0, wsem1)
    csem = (csem0, csem1)
    rows_lo = (n_nodes // NS) // 8 * 8
    tail = n_nodes - NS * rows_lo

    zero = jnp.zeros((L,), jnp.float32)

    def zb(i, _):
        w_v[0, i // nsub, pl.ds((i % nsub) * L, L)] = zero
        return 0

    lax.fori_loop(0, CH * nsub, zb, 0)
    base_r = pl.multiple_of(sid * rows_lo, 8)
    nfull = rows_lo // CH
    rem = rows_lo - nfull * CH
    for k in range(nfull):
        pltpu.sync_copy(w_v.at[0], agg_sh.at[pl.ds(base_r + k * CH, CH)])
    if rem:
        pltpu.sync_copy(w_v.at[0, pl.ds(0, rem)],
                        agg_sh.at[pl.ds(base_r + nfull * CH, rem)])
    if tail:
        @pl.when(sid == NS - 1)
        def _zero_tail():
            pltpu.sync_copy(w_v.at[0, pl.ds(0, tail)],
                            agg_sh.at[pl.ds(NS * rows_lo, tail)])
    plsc.subcore_barrier()

    def sidx_desc(ci, p):
        base = pl.multiple_of(n_edges + (wid * per_tile + ci) * CH, 8)
        return pltpu.make_async_copy(
            ei_hbm.at[pl.ds(base, CH)], src_v.at[p], ssem[p])

    def didx_desc(ci, p):
        base = pl.multiple_of((wid * per_tile + ci) * CH, 8)
        return pltpu.make_async_copy(
            ei_hbm.at[pl.ds(base, CH)], dst_v.at[p], dsem[p])

    def g_desc(p):
        return pltpu.make_async_copy(
            h1_hbm.at[src_v.at[p]], rows_v.at[p], gsem[p])

    def w_desc(ci, p):
        base = pl.multiple_of((wid * per_tile + ci) * CH, 8)
        return pltpu.make_async_copy(
            w_hbm.at[pl.ds(base, CH)], w_v.at[p], wsem[p])

    def c_desc(ci, p):
        base = pl.multiple_of((wid * per_tile + ci) * CH, 8)
        return pltpu.make_async_copy(
            c_hbm.at[pl.ds(base, CH)], c_v.at[p, pl.ds(0, CH)], csem[p])

    sidx_desc(0, 0).start()
    didx_desc(0, 0).start()
    sidx_desc(1, 1).start()
    didx_desc(1, 1).start()
    sidx_desc(0, 0).wait()
    g_desc(0).start()
    w_desc(0, 0).start()
    c_desc(0, 0).start()

    def step(ci, p):
        q = 1 - p

        @pl.when(ci + 1 < per_tile)
        def _prefetch_gw():
            sidx_desc(ci + 1, q).wait()
            g_desc(q).start()
            w_desc(ci + 1, q).start()
            c_desc(ci + 1, q).start()

        g_desc(p).wait()
        w_desc(ci, p).wait()
        c_desc(ci, p).wait()
        rvp = rows_v.at[p]
        wvp = w_v.at[p]
        cvp = c_v.at[p]

        @plsc.parallel_loop(0, CH, step=1, unroll=4)
        def _mul(ei):
            cs = cvp[pl.ds(ei, L)][0]
            for j in range(nsub):
                s = pl.ds(j * L, L)
                rvp[ei, s] = rvp[ei, s] * (wvp[ei, s] * cs)

        didx_desc(ci, p).wait()
        pltpu.sync_copy(rvp, agg_sh.at[dst_v.at[p]], add=True)

        @pl.when(ci + 2 < per_tile)
        def _prefetch_idx():
            sidx_desc(ci + 2, p).start()
            didx_desc(ci + 2, p).start()

    def pair_body(k, _):
        step(2 * k, 0)
        step(2 * k + 1, 1)
        return 0

    lax.fori_loop(0, per_tile // 2, pair_body, 0)
    if per_tile % 2:
        step(per_tile - 1, (per_tile - 1) % 2)
    plsc.subcore_barrier()

    out_base = pl.multiple_of(cid * n_nodes + sid * rows_lo, 8)
    pltpu.sync_copy(agg_sh.at[pl.ds(base_r, rows_lo)],
                    out_hbm.at[pl.ds(out_base, rows_lo)])
    if tail:
        @pl.when(sid == NS - 1)
        def _write_tail():
            tb = pl.multiple_of(cid * n_nodes + NS * rows_lo, 8)
            pltpu.sync_copy(agg_sh.at[pl.ds(NS * rows_lo, tail)],
                            out_hbm.at[pl.ds(tb, tail)])


def kernel(h, edge_index, edge_weight, edge_attr, lin1_w, lin2_w, lin2_b,
           mlp_w1, mlp_b1, mlp_w2, mlp_b2):
    n, d = h.shape
    e = edge_weight.shape[0]
    ng = edge_attr.shape[1]
    nf = mlp_w1.shape[0]
    assert e % (NW * CH) == 0
    per_tile = e // (NW * CH)

    ei_flat = edge_index.reshape(2 * e)
    b1 = mlp_b1.reshape(1, nf)
    b2 = mlp_b2.reshape(1, nf)
    bo = lin2_b.reshape(1, d)

    rb = 2000
    h1 = pl.pallas_call(
        _lin_body,
        grid=(n // rb,),
        in_specs=[pl.BlockSpec((rb, d), lambda i: (i, 0)),
                  pl.BlockSpec((nf, d), lambda i: (0, 0))],
        out_specs=pl.BlockSpec((rb, nf), lambda i: (i, 0)),
        out_shape=jax.ShapeDtypeStruct((n, nf), jnp.float32),
    )(h, lin1_w)

    ewv = edge_weight.reshape(e // 128, 128)
    c_edges = pl.pallas_call(
        _cutoff_body,
        out_shape=jax.ShapeDtypeStruct((e // 128, 128), jnp.float32),
    )(ewv)
    c_flat = c_edges.reshape(e)

    eb = 2560
    attr_bf = edge_attr.astype(jnp.bfloat16)
    w1_bf = mlp_w1.astype(jnp.bfloat16)
    w2_bf = mlp_w2.astype(jnp.bfloat16)
    w_edges = pl.pallas_call(
        _filter_body,
        grid=(e // eb,),
        in_specs=[pl.BlockSpec((eb, ng), lambda i: (i, 0)),
                  pl.BlockSpec((nf, ng), lambda i: (0, 0)),
                  pl.BlockSpec((1, nf), lambda i: (0, 0)),
                  pl.BlockSpec((nf, nf), lambda i: (0, 0)),
                  pl.BlockSpec((1, nf), lambda i: (0, 0))],
        out_specs=pl.BlockSpec((eb, nf), lambda i: (i, 0)),
        out_shape=jax.ShapeDtypeStruct((e, nf), jnp.float32),
    )(attr_bf, w1_bf, b1, w2_bf, b2)

    mesh = plsc.VectorSubcoreMesh(core_axis_name="c", subcore_axis_name="s")
    partials = pl.kernel(
        functools.partial(_sc_gather_mul_scatter, n, d, per_tile, e),
        out_type=jax.ShapeDtypeStruct((NC * n, d), jnp.float32),
        mesh=mesh,
        scratch_types=[
            pltpu.VMEM((2, CH), jnp.int32),
            pltpu.VMEM((2, CH), jnp.int32),
            pltpu.VMEM((2, CH + L), jnp.float32),
            pltpu.VMEM((2, CH, d), jnp.float32),
            pltpu.VMEM((2, CH, d), jnp.float32),
            pltpu.VMEM_SHARED((n, d), jnp.float32),
            pltpu.SemaphoreType.DMA,
            pltpu.SemaphoreType.DMA,
            pltpu.SemaphoreType.DMA,
            pltpu.SemaphoreType.DMA,
            pltpu.SemaphoreType.DMA,
            pltpu.SemaphoreType.DMA,
            pltpu.SemaphoreType.DMA,
            pltpu.SemaphoreType.DMA,
            pltpu.SemaphoreType.DMA,
            pltpu.SemaphoreType.DMA,
        ],
    )(h1, ei_flat, w_edges, c_flat)

    nb = n // rb
    out = pl.pallas_call(
        _out_body,
        grid=(nb,),
        in_specs=[pl.BlockSpec((rb, nf), lambda i: (i, 0)),
                  pl.BlockSpec((rb, nf), lambda i: (i, 0)),
                  pl.BlockSpec((rb, nf), lambda i: (i + nb, 0)),
                  pl.BlockSpec((d, nf), lambda i: (0, 0)),
                  pl.BlockSpec((1, d), lambda i: (0, 0))],
        out_specs=pl.BlockSpec((rb, d), lambda i: (i, 0)),
        out_shape=jax.ShapeDtypeStruct((n, d), jnp.float32),
    )(h1, partials, partials, lin2_w, bo)

    return out


# speedup vs baseline: 4.1582x; 1.0687x over previous
import functools

import jax
import jax.numpy as jnp
import numpy as np
from jax import lax
from jax.experimental import pallas as pl
from jax.experimental.pallas import tpu as pltpu
from jax.experimental.pallas import tpu_sc as plsc

CUTOFF = 10.0
LOG2 = float(np.log(2.0))
PI = float(np.pi)

NC = 2
NS = 16
L = 16
NW = NC * NS
CH = 80


def _lin_body(x_ref, w_ref, o_ref):
    o_ref[...] = lax.dot_general(
        x_ref[...], w_ref[...], (((1,), (1,)), ((), ())),
        preferred_element_type=jnp.float32)


def _filter_body(attr_ref, w1_ref, b1_ref, w2_ref, b2_ref, o_ref):
    x = lax.dot_general(attr_ref[...], w1_ref[...], (((1,), (1,)), ((), ())),
                        preferred_element_type=jnp.float32)
    x = x + b1_ref[...]
    x = jnp.maximum(x, 0.0) + jnp.log(1.0 + jnp.exp(-jnp.abs(x))) - LOG2
    x = lax.dot_general(x.astype(jnp.bfloat16), w2_ref[...],
                        (((1,), (1,)), ((), ())),
                        preferred_element_type=jnp.float32)
    o_ref[...] = x + b2_ref[...]


def _cutoff_body(ew_ref, c_ref):
    c_ref[...] = 0.5 * (jnp.cos(ew_ref[...] * (PI / CUTOFF)) + 1.0)


def _out_body(h1_ref, a0_ref, a1_ref, w_ref, b_ref, o_ref):
    s = h1_ref[...] + a0_ref[...] + a1_ref[...]
    o_ref[...] = lax.dot_general(
        s, w_ref[...], (((1,), (1,)), ((), ())),
        preferred_element_type=jnp.float32) + b_ref[...]


def _sc_gather_mul_scatter(n_nodes, d, per_tile, n_edges,
                           h1_hbm, ei_hbm, w_hbm, c_hbm, out_hbm,
                           src_v, dst_v, c_v, rows_v, w_v, agg_sh,
                           ssem0, ssem1, dsem0, dsem1,
                           gsem0, gsem1, wsem0, wsem1, csem0, csem1,
                           asem0, asem1):
    cid = lax.axis_index("c")
    sid = lax.axis_index("s")
    wid = cid * NS + sid
    nsub = d // L
    ssem = (ssem0, ssem1)
    dsem = (dsem0, dsem1)
    gsem = (gsem0, gsem1)
    wsem = (wsem0, wsem1)
    csem = (csem0, csem1)
    asem = (asem0, asem1)
    rows_lo = (n_nodes // NS) // 8 * 8
    tail = n_nodes - NS * rows_lo

    zero = jnp.zeros((L,), jnp.float32)

    def zb(i, _):
        w_v[0, i // nsub, pl.ds((i % nsub) * L, L)] = zero
        return 0

    lax.fori_loop(0, CH * nsub, zb, 0)
    base_r = pl.multiple_of(sid * rows_lo, 8)
    nfull = rows_lo // CH
    rem = rows_lo - nfull * CH
    for k in range(nfull):
        pltpu.sync_copy(w_v.at[0], agg_sh.at[pl.ds(base_r + k * CH, CH)])
    if rem:
        pltpu.sync_copy(w_v.at[0, pl.ds(0, rem)],
                        agg_sh.at[pl.ds(base_r + nfull * CH, rem)])
    if tail:
        @pl.when(sid == NS - 1)
        def _zero_tail():
            pltpu.sync_copy(w_v.at[0, pl.ds(0, tail)],
                            agg_sh.at[pl.ds(NS * rows_lo, tail)])
    plsc.subcore_barrier()

    def sidx_desc(ci, p):
        base = pl.multiple_of(n_edges + (wid * per_tile + ci) * CH, 8)
        return pltpu.make_async_copy(
            ei_hbm.at[pl.ds(base, CH)], src_v.at[p], ssem[p])

    def didx_desc(ci, p):
        base = pl.multiple_of((wid * per_tile + ci) * CH, 8)
        return pltpu.make_async_copy(
            ei_hbm.at[pl.ds(base, CH)], dst_v.at[p], dsem[p])

    def g_desc(p):
        return pltpu.make_async_copy(
            h1_hbm.at[src_v.at[p]], rows_v.at[p], gsem[p])

    def w_desc(ci, p):
        base = pl.multiple_of((wid * per_tile + ci) * CH, 8)
        return pltpu.make_async_copy(
            w_hbm.at[pl.ds(base, CH)], w_v.at[p], wsem[p])

    def c_desc(ci, p):
        base = pl.multiple_of((wid * per_tile + ci) * CH, 8)
        return pltpu.make_async_copy(
            c_hbm.at[pl.ds(base, CH)], c_v.at[p, pl.ds(0, CH)], csem[p])

    def a_start(p):
        pltpu.async_copy(rows_v.at[p], agg_sh.at[dst_v.at[p]], asem[p],
                         add=True)

    def a_wait(p):
        pltpu.make_async_copy(
            rows_v.at[p], agg_sh.at[dst_v.at[p]], asem[p]).wait()

    sidx_desc(0, 0).start()
    didx_desc(0, 0).start()
    sidx_desc(1, 1).start()
    sidx_desc(0, 0).wait()
    g_desc(0).start()
    w_desc(0, 0).start()
    c_desc(0, 0).start()

    def step(ci, p):
        q = 1 - p

        @pl.when(ci + 1 < per_tile)
        def _prefetch_gw():
            @pl.when(ci >= 1)
            def _drain_prev_scatter():
                a_wait(q)
            didx_desc(ci + 1, q).start()
            sidx_desc(ci + 1, q).wait()
            g_desc(q).start()
            w_desc(ci + 1, q).start()
            c_desc(ci + 1, q).start()

        g_desc(p).wait()
        w_desc(ci, p).wait()
        c_desc(ci, p).wait()

        @pl.when(ci + 2 < per_tile)
        def _prefetch_sidx():
            sidx_desc(ci + 2, p).start()

        rvp = rows_v.at[p]
        wvp = w_v.at[p]
        cvp = c_v.at[p]

        @plsc.parallel_loop(0, CH, step=1, unroll=4)
        def _mul(ei):
            cs = cvp[pl.ds(ei, L)][0]
            for j in range(nsub):
                s = pl.ds(j * L, L)
                rvp[ei, s] = rvp[ei, s] * (wvp[ei, s] * cs)

        didx_desc(ci, p).wait()
        a_start(p)

    def pair_body(k, _):
        step(2 * k, 0)
        step(2 * k + 1, 1)
        return 0

    lax.fori_loop(0, per_tile // 2, pair_body, 0)
    if per_tile % 2:
        step(per_tile - 1, (per_tile - 1) % 2)
    a_wait((per_tile - 2) % 2)
    a_wait((per_tile - 1) % 2)
    plsc.subcore_barrier()

    out_base = pl.multiple_of(cid * n_nodes + sid * rows_lo, 8)
    pltpu.sync_copy(agg_sh.at[pl.ds(base_r, rows_lo)],
                    out_hbm.at[pl.ds(out_base, rows_lo)])
    if tail:
        @pl.when(sid == NS - 1)
        def _write_tail():
            tb = pl.multiple_of(cid * n_nodes + NS * rows_lo, 8)
            pltpu.sync_copy(agg_sh.at[pl.ds(NS * rows_lo, tail)],
                            out_hbm.at[pl.ds(tb, tail)])


def kernel(h, edge_index, edge_weight, edge_attr, lin1_w, lin2_w, lin2_b,
           mlp_w1, mlp_b1, mlp_w2, mlp_b2):
    n, d = h.shape
    e = edge_weight.shape[0]
    ng = edge_attr.shape[1]
    nf = mlp_w1.shape[0]
    assert e % (NW * CH) == 0
    per_tile = e // (NW * CH)

    ei_flat = edge_index.reshape(2 * e)
    b1 = mlp_b1.reshape(1, nf)
    b2 = mlp_b2.reshape(1, nf)
    bo = lin2_b.reshape(1, d)

    rb = 2000
    h1 = pl.pallas_call(
        _lin_body,
        grid=(n // rb,),
        in_specs=[pl.BlockSpec((rb, d), lambda i: (i, 0)),
                  pl.BlockSpec((nf, d), lambda i: (0, 0))],
        out_specs=pl.BlockSpec((rb, nf), lambda i: (i, 0)),
        out_shape=jax.ShapeDtypeStruct((n, nf), jnp.float32),
    )(h, lin1_w)

    ewv = edge_weight.reshape(e // 128, 128)
    c_edges = pl.pallas_call(
        _cutoff_body,
        out_shape=jax.ShapeDtypeStruct((e // 128, 128), jnp.float32),
    )(ewv)
    c_flat = c_edges.reshape(e)

    eb = 2560
    attr_bf = edge_attr.astype(jnp.bfloat16)
    w1_bf = mlp_w1.astype(jnp.bfloat16)
    w2_bf = mlp_w2.astype(jnp.bfloat16)
    w_edges = pl.pallas_call(
        _filter_body,
        grid=(e // eb,),
        in_specs=[pl.BlockSpec((eb, ng), lambda i: (i, 0)),
                  pl.BlockSpec((nf, ng), lambda i: (0, 0)),
                  pl.BlockSpec((1, nf), lambda i: (0, 0)),
                  pl.BlockSpec((nf, nf), lambda i: (0, 0)),
                  pl.BlockSpec((1, nf), lambda i: (0, 0))],
        out_specs=pl.BlockSpec((eb, nf), lambda i: (i, 0)),
        out_shape=jax.ShapeDtypeStruct((e, nf), jnp.float32),
    )(attr_bf, w1_bf, b1, w2_bf, b2)

    mesh = plsc.VectorSubcoreMesh(core_axis_name="c", subcore_axis_name="s")
    partials = pl.kernel(
        functools.partial(_sc_gather_mul_scatter, n, d, per_tile, e),
        out_type=jax.ShapeDtypeStruct((NC * n, d), jnp.float32),
        mesh=mesh,
        scratch_types=[
            pltpu.VMEM((2, CH), jnp.int32),
            pltpu.VMEM((2, CH), jnp.int32),
            pltpu.VMEM((2, CH + L), jnp.float32),
            pltpu.VMEM((2, CH, d), jnp.float32),
            pltpu.VMEM((2, CH, d), jnp.float32),
            pltpu.VMEM_SHARED((n, d), jnp.float32),
            pltpu.SemaphoreType.DMA,
            pltpu.SemaphoreType.DMA,
            pltpu.SemaphoreType.DMA,
            pltpu.SemaphoreType.DMA,
            pltpu.SemaphoreType.DMA,
            pltpu.SemaphoreType.DMA,
            pltpu.SemaphoreType.DMA,
            pltpu.SemaphoreType.DMA,
            pltpu.SemaphoreType.DMA,
            pltpu.SemaphoreType.DMA,
            pltpu.SemaphoreType.DMA,
            pltpu.SemaphoreType.DMA,
        ],
    )(h1, ei_flat, w_edges, c_flat)

    nb = n // rb
    out = pl.pallas_call(
        _out_body,
        grid=(nb,),
        in_specs=[pl.BlockSpec((rb, nf), lambda i: (i, 0)),
                  pl.BlockSpec((rb, nf), lambda i: (i, 0)),
                  pl.BlockSpec((rb, nf), lambda i: (i + nb, 0)),
                  pl.BlockSpec((d, nf), lambda i: (0, 0)),
                  pl.BlockSpec((1, d), lambda i: (0, 0))],
        out_specs=pl.BlockSpec((rb, d), lambda i: (i, 0)),
        out_shape=jax.ShapeDtypeStruct((n, d), jnp.float32),
    )(h1, partials, partials, lin2_w, bo)

    return out


# speedup vs baseline: 4.8212x; 1.1594x over previous
import functools

import jax
import jax.numpy as jnp
import numpy as np
from jax import lax
from jax.experimental import pallas as pl
from jax.experimental.pallas import tpu as pltpu
from jax.experimental.pallas import tpu_sc as plsc

CUTOFF = 10.0
LOG2 = float(np.log(2.0))
PI = float(np.pi)

NC = 2
NS = 16
L = 16
NW = NC * NS
CH = 80


def _lin_body(x_ref, w_ref, o_ref):
    o_ref[...] = lax.dot_general(
        x_ref[...], w_ref[...], (((1,), (1,)), ((), ())),
        preferred_element_type=jnp.float32)


def _filter_body(attr_ref, w1_ref, b1_ref, w2_ref, b2_ref, o_ref):
    x = lax.dot_general(attr_ref[...], w1_ref[...], (((1,), (1,)), ((), ())),
                        preferred_element_type=jnp.float32)
    x = x + b1_ref[...]
    x = jnp.maximum(x, 0.0) + jnp.log(1.0 + jnp.exp(-jnp.abs(x))) - LOG2
    x = lax.dot_general(x.astype(jnp.bfloat16), w2_ref[...],
                        (((1,), (1,)), ((), ())),
                        preferred_element_type=jnp.float32)
    o_ref[...] = x + b2_ref[...]


def _cutoff_body(ew_ref, c_ref):
    c_ref[...] = 0.5 * (jnp.cos(ew_ref[...] * (PI / CUTOFF)) + 1.0)


def _out_body(h1_ref, a0_ref, a1_ref, b0_ref, b1_ref, w_ref, b_ref, o_ref):
    s = (h1_ref[...] + a0_ref[...] + a1_ref[...]
         + b0_ref[...] + b1_ref[...])
    o_ref[...] = lax.dot_general(
        s, w_ref[...], (((1,), (1,)), ((), ())),
        preferred_element_type=jnp.float32) + b_ref[...]


def _sc_gather_mul_scatter(n_nodes, d, per_tile, n_edges, e_off,
                           h1_hbm, ei_hbm, w_hbm, c_hbm, out_hbm,
                           src_v, dst_v, c_v, rows_v, w_v, agg_sh,
                           ssem0, ssem1, dsem0, dsem1,
                           gsem0, gsem1, wsem0, wsem1, csem0, csem1,
                           asem0, asem1):
    cid = lax.axis_index("c")
    sid = lax.axis_index("s")
    wid = cid * NS + sid
    nsub = d // L
    ssem = (ssem0, ssem1)
    dsem = (dsem0, dsem1)
    gsem = (gsem0, gsem1)
    wsem = (wsem0, wsem1)
    csem = (csem0, csem1)
    asem = (asem0, asem1)
    rows_lo = (n_nodes // NS) // 8 * 8
    tail = n_nodes - NS * rows_lo

    zero = jnp.zeros((L,), jnp.float32)

    def zb(i, _):
        w_v[0, i // nsub, pl.ds((i % nsub) * L, L)] = zero
        return 0

    lax.fori_loop(0, CH * nsub, zb, 0)
    base_r = pl.multiple_of(sid * rows_lo, 8)
    nfull = rows_lo // CH
    rem = rows_lo - nfull * CH
    for k in range(nfull):
        pltpu.sync_copy(w_v.at[0], agg_sh.at[pl.ds(base_r + k * CH, CH)])
    if rem:
        pltpu.sync_copy(w_v.at[0, pl.ds(0, rem)],
                        agg_sh.at[pl.ds(base_r + nfull * CH, rem)])
    if tail:
        @pl.when(sid == NS - 1)
        def _zero_tail():
            pltpu.sync_copy(w_v.at[0, pl.ds(0, tail)],
                            agg_sh.at[pl.ds(NS * rows_lo, tail)])
    plsc.subcore_barrier()

    def sidx_desc(ci, p):
        base = pl.multiple_of(n_edges + e_off + (wid * per_tile + ci) * CH, 8)
        return pltpu.make_async_copy(
            ei_hbm.at[pl.ds(base, CH)], src_v.at[p], ssem[p])

    def didx_desc(ci, p):
        base = pl.multiple_of(e_off + (wid * per_tile + ci) * CH, 8)
        return pltpu.make_async_copy(
            ei_hbm.at[pl.ds(base, CH)], dst_v.at[p], dsem[p])

    def g_desc(p):
        return pltpu.make_async_copy(
            h1_hbm.at[src_v.at[p]], rows_v.at[p], gsem[p])

    def w_desc(ci, p):
        base = pl.multiple_of((wid * per_tile + ci) * CH, 8)
        return pltpu.make_async_copy(
            w_hbm.at[pl.ds(base, CH)], w_v.at[p], wsem[p])

    def c_desc(ci, p):
        base = pl.multiple_of(e_off + (wid * per_tile + ci) * CH, 8)
        return pltpu.make_async_copy(
            c_hbm.at[pl.ds(base, CH)], c_v.at[p, pl.ds(0, CH)], csem[p])

    def a_start(p):
        pltpu.async_copy(rows_v.at[p], agg_sh.at[dst_v.at[p]], asem[p],
                         add=True)

    def a_wait(p):
        pltpu.make_async_copy(
            rows_v.at[p], agg_sh.at[dst_v.at[p]], asem[p]).wait()

    sidx_desc(0, 0).start()
    didx_desc(0, 0).start()
    sidx_desc(1, 1).start()
    sidx_desc(0, 0).wait()
    g_desc(0).start()
    w_desc(0, 0).start()
    c_desc(0, 0).start()

    def step(ci, p):
        q = 1 - p

        @pl.when(ci + 1 < per_tile)
        def _prefetch_gw():
            @pl.when(ci >= 1)
            def _drain_prev_scatter():
                a_wait(q)
            didx_desc(ci + 1, q).start()
            sidx_desc(ci + 1, q).wait()
            g_desc(q).start()
            w_desc(ci + 1, q).start()
            c_desc(ci + 1, q).start()

        g_desc(p).wait()
        w_desc(ci, p).wait()
        c_desc(ci, p).wait()

        @pl.when(ci + 2 < per_tile)
        def _prefetch_sidx():
            sidx_desc(ci + 2, p).start()

        rvp = rows_v.at[p]
        wvp = w_v.at[p]
        cvp = c_v.at[p]

        @plsc.parallel_loop(0, CH, step=1, unroll=4)
        def _mul(ei):
            cs = cvp[pl.ds(ei, L)][0]
            for j in range(nsub):
                s = pl.ds(j * L, L)
                rvp[ei, s] = rvp[ei, s] * (wvp[ei, s] * cs)

        didx_desc(ci, p).wait()
        a_start(p)

    def pair_body(k, _):
        step(2 * k, 0)
        step(2 * k + 1, 1)
        return 0

    lax.fori_loop(0, per_tile // 2, pair_body, 0)
    if per_tile % 2:
        step(per_tile - 1, (per_tile - 1) % 2)
    a_wait((per_tile - 2) % 2)
    a_wait((per_tile - 1) % 2)
    plsc.subcore_barrier()

    out_base = pl.multiple_of(cid * n_nodes + sid * rows_lo, 8)
    pltpu.sync_copy(agg_sh.at[pl.ds(base_r, rows_lo)],
                    out_hbm.at[pl.ds(out_base, rows_lo)])
    if tail:
        @pl.when(sid == NS - 1)
        def _write_tail():
            tb = pl.multiple_of(cid * n_nodes + NS * rows_lo, 8)
            pltpu.sync_copy(agg_sh.at[pl.ds(NS * rows_lo, tail)],
                            out_hbm.at[pl.ds(tb, tail)])


def kernel(h, edge_index, edge_weight, edge_attr, lin1_w, lin2_w, lin2_b,
           mlp_w1, mlp_b1, mlp_w2, mlp_b2):
    n, d = h.shape
    e = edge_weight.shape[0]
    ng = edge_attr.shape[1]
    nf = mlp_w1.shape[0]
    assert e % (NW * CH) == 0
    per_tile = e // (NW * CH)

    ei_flat = edge_index.reshape(2 * e)
    b1 = mlp_b1.reshape(1, nf)
    b2 = mlp_b2.reshape(1, nf)
    bo = lin2_b.reshape(1, d)

    rb = 2000
    h1 = pl.pallas_call(
        _lin_body,
        grid=(n // rb,),
        in_specs=[pl.BlockSpec((rb, d), lambda i: (i, 0)),
                  pl.BlockSpec((nf, d), lambda i: (0, 0))],
        out_specs=pl.BlockSpec((rb, nf), lambda i: (i, 0)),
        out_shape=jax.ShapeDtypeStruct((n, nf), jnp.float32),
    )(h, lin1_w)

    ewv = edge_weight.reshape(e // 128, 128)
    c_edges = pl.pallas_call(
        _cutoff_body,
        out_shape=jax.ShapeDtypeStruct((e // 128, 128), jnp.float32),
    )(ewv)
    c_flat = c_edges.reshape(e)

    eb = 2560
    attr_bf = edge_attr.astype(jnp.bfloat16)
    w1_bf = mlp_w1.astype(jnp.bfloat16)
    w2_bf = mlp_w2.astype(jnp.bfloat16)
    mesh = plsc.VectorSubcoreMesh(core_axis_name="c", subcore_axis_name="s")
    seg_tiles = (per_tile // 2, per_tile - per_tile // 2)
    partial_list = []
    e_off = 0
    for kseg in seg_tiles:
        seg_e = kseg * NW * CH
        off_blk = e_off // eb

        w_seg = pl.pallas_call(
            _filter_body,
            grid=(seg_e // eb,),
            in_specs=[
                pl.BlockSpec((eb, ng), lambda i, ob=off_blk: (i + ob, 0)),
                pl.BlockSpec((nf, ng), lambda i: (0, 0)),
                pl.BlockSpec((1, nf), lambda i: (0, 0)),
                pl.BlockSpec((nf, nf), lambda i: (0, 0)),
                pl.BlockSpec((1, nf), lambda i: (0, 0))],
            out_specs=pl.BlockSpec((eb, nf), lambda i: (i, 0)),
            out_shape=jax.ShapeDtypeStruct((seg_e, nf), jnp.float32),
        )(attr_bf, w1_bf, b1, w2_bf, b2)

        partial_list.append(pl.kernel(
            functools.partial(_sc_gather_mul_scatter, n, d, kseg, e, e_off),
            out_type=jax.ShapeDtypeStruct((NC * n, d), jnp.float32),
            mesh=mesh,
            scratch_types=[
                pltpu.VMEM((2, CH), jnp.int32),
                pltpu.VMEM((2, CH), jnp.int32),
                pltpu.VMEM((2, CH + L), jnp.float32),
                pltpu.VMEM((2, CH, d), jnp.float32),
                pltpu.VMEM((2, CH, d), jnp.float32),
                pltpu.VMEM_SHARED((n, d), jnp.float32),
                pltpu.SemaphoreType.DMA,
                pltpu.SemaphoreType.DMA,
                pltpu.SemaphoreType.DMA,
                pltpu.SemaphoreType.DMA,
                pltpu.SemaphoreType.DMA,
                pltpu.SemaphoreType.DMA,
                pltpu.SemaphoreType.DMA,
                pltpu.SemaphoreType.DMA,
                pltpu.SemaphoreType.DMA,
                pltpu.SemaphoreType.DMA,
                pltpu.SemaphoreType.DMA,
                pltpu.SemaphoreType.DMA,
            ],
        )(h1, ei_flat, w_seg, c_flat))
        e_off += seg_e

    nb = n // rb
    p0, p1 = partial_list
    out = pl.pallas_call(
        _out_body,
        grid=(nb,),
        in_specs=[pl.BlockSpec((rb, nf), lambda i: (i, 0)),
                  pl.BlockSpec((rb, nf), lambda i: (i, 0)),
                  pl.BlockSpec((rb, nf), lambda i: (i + nb, 0)),
                  pl.BlockSpec((rb, nf), lambda i: (i, 0)),
                  pl.BlockSpec((rb, nf), lambda i: (i + nb, 0)),
                  pl.BlockSpec((d, nf), lambda i: (0, 0)),
                  pl.BlockSpec((1, d), lambda i: (0, 0))],
        out_specs=pl.BlockSpec((rb, d), lambda i: (i, 0)),
        out_shape=jax.ShapeDtypeStruct((n, d), jnp.float32),
    )(h1, p0, p0, p1, p1, lin2_w, bo)

    return out
